# baseline (speedup 1.0000x reference)
import functools

import jax
import jax.numpy as jnp
from jax import lax
from jax.experimental import pallas as pl
from jax.experimental.pallas import tpu as pltpu
from jax.experimental.pallas import tpu_sc as plsc

B, H, S, D, DM = 8, 16, 577, 1024, 64
DOM = 54
CTX = 10
KEEP = S - (DOM + 1)
STEP = max(1, KEEP // CTX)
NSEL = DOM + 1
OUT_T = NSEL + CTX
GSLOT = 64
NROWS = B * GSLOT

_HIGH = lax.Precision.HIGHEST
_DEF = lax.Precision.DEFAULT


def _mm_tn(a, bm, prec=_DEF):
    return lax.dot_general(a, bm, (((0,), (0,)), ((), ())), precision=prec)


def _mm(a, bm, prec=_DEF):
    return lax.dot_general(a, bm, (((1,), (0,)), ((), ())), precision=prec)


def _tc1_select(attn_cls_ref, sel_ref, idx_ref, gidx_ref):
    f32 = jnp.float32
    b = pl.program_id(0)

    ii = lax.broadcasted_iota(jnp.int32, (S, S), 0)
    jj = lax.broadcasted_iota(jnp.int32, (S, S), 1)

    attn = attn_cls_ref[b]
    v = jnp.sum(attn, axis=0, keepdims=True)
    col0 = lax.broadcasted_iota(jnp.int32, (1, S), 1) == 0
    v = jnp.where(col0, -jnp.inf, v)
    vcol = _mm_tn(v, jnp.ones((1, 1), f32), _HIGH)

    vi = jnp.broadcast_to(vcol, (S, S))
    vj = jnp.broadcast_to(v, (S, S))
    beats = (vi > vj) | ((vi == vj) & (ii < jj))
    nbeat = jnp.sum(beats.astype(jnp.int32), axis=1, keepdims=True)
    rank = (S - 1) - nbeat
    selc = rank < DOM

    pp = lax.broadcasted_iota(jnp.int32, (S, 128), 1)
    in_slot = ((jnp.broadcast_to(rank, (S, 128)) == pp - 1)
               & jnp.broadcast_to(selc, (S, 128)))
    ii128 = lax.broadcasted_iota(jnp.int32, (S, 128), 0)
    idx_row = jnp.sum(jnp.where(in_slot, ii128, 0), axis=0,
                      keepdims=True)
    idx_ref[b, 0:1, :] = idx_row

    icol = lax.broadcasted_iota(jnp.int32, (S, 1), 0)
    sel_col = (selc | (icol == 0)).astype(f32)
    lane = lax.broadcasted_iota(jnp.int32, (S, B), 1)

    @pl.when(b == 0)
    def _init():
        sel_ref[:, :] = jnp.zeros((S, B), f32)

    sel_ref[:, :] += jnp.where(lane == b, jnp.broadcast_to(sel_col, (S, B)),
                               0.0)
    ltT = (jj < ii).astype(f32)
    dom_rank = _mm(ltT, sel_col)

    dri = dom_rank.astype(jnp.int32)
    gslot = lax.broadcasted_iota(jnp.int32, (S, GSLOT), 1)
    dom_slot = ((jnp.broadcast_to(dri, (S, GSLOT)) == gslot)
                & jnp.broadcast_to(sel_col > 0.5, (S, GSLOT)))
    iig = lax.broadcasted_iota(jnp.int32, (S, GSLOT), 0)
    gidx = jnp.sum(jnp.where(dom_slot, iig * B + b, 0), axis=0,
                   keepdims=True)
    gidx_ref[pl.ds(b, 1), :] = gidx


def _tc2_merge(sel_ref, hid_ref, met_ref, ctx_ref):
    f32 = jnp.float32
    b = pl.program_id(0)

    ii = lax.broadcasted_iota(jnp.int32, (S, S), 0)
    jj = lax.broadcasted_iota(jnp.int32, (S, S), 1)

    lane = lax.broadcasted_iota(jnp.int32, (S, B), 1)
    sel_col = jnp.sum(jnp.where(lane == b, sel_ref[:, :], 0.0), axis=1,
                      keepdims=True)
    ltT = (jj < ii).astype(f32)
    dom_rank = _mm(ltT, sel_col)
    icol = lax.broadcasted_iota(jnp.int32, (S, 1), 0)
    unsel = 1.0 - sel_col
    f_rank = icol.astype(f32) - dom_rank

    t_id = jnp.floor((f_rank + 0.5) * (1.0 / STEP))
    is_target = ((unsel > 0.5) & (f_rank == t_id * float(STEP))
                 & (f_rank < float(STEP * CTX)))
    is_merge = (unsel > 0.5) & (~is_target)

    mt = met_ref[b]
    ss = jnp.sum(mt * mt, axis=0, keepdims=True)
    mn_t = mt / jnp.sqrt(ss)
    trow = lax.broadcasted_iota(jnp.int32, (S, 16), 1).astype(f32)
    tsel = ((jnp.broadcast_to(t_id, (S, 16)) == trow)
            & jnp.broadcast_to(is_target, (S, 16))).astype(f32)
    tmat_t = _mm(mn_t, tsel, _HIGH)
    sim = _mm_tn(mn_t, tmat_t)
    sim = jnp.where(trow < float(CTX), sim, -jnp.inf)
    mx = jnp.max(sim, axis=1, keepdims=True)
    assign = jnp.min(jnp.where(sim == mx, trow, 1e9), axis=1,
                     keepdims=True)
    amat = ((jnp.broadcast_to(assign, (S, 16)) == trow)
            & jnp.broadcast_to(is_merge, (S, 16))).astype(f32)
    counts = jnp.sum(amat, axis=0, keepdims=True)
    inv_counts = 1.0 / jnp.maximum(counts, 1.0)
    ct = tsel + amat * jnp.broadcast_to(inv_counts, (S, 16))

    hid = hid_ref[:, b, :]
    out_ctx = _mm_tn(ct, hid)
    ctx_ref[:, b, :] = out_ctx


_info = plsc.get_sparse_core_info()
_NC, _NS = _info.num_cores, _info.num_subcores
_RPW = NROWS // (_NC * _NS)

_sc_mesh = plsc.VectorSubcoreMesh(core_axis_name="c", subcore_axis_name="s")


@functools.partial(
    pl.kernel,
    mesh=_sc_mesh,
    out_type=jax.ShapeDtypeStruct((NROWS, D), jnp.float32),
    scratch_types=[
        pltpu.VMEM((_RPW,), jnp.int32),
        pltpu.VMEM((_RPW, D), jnp.float32),
        pltpu.SemaphoreType.DMA,
    ],
)
def _sc_gather(gidx_hbm, table_hbm, out_hbm, idx_v, rows_v, sem):
    wid = lax.axis_index("s") * _NC + lax.axis_index("c")
    base = wid * _RPW
    pltpu.sync_copy(gidx_hbm.at[pl.ds(base, _RPW)], idx_v)
    pltpu.async_copy(table_hbm.at[idx_v], rows_v, sem).wait()
    pltpu.sync_copy(rows_v, out_hbm.at[pl.ds(base, _RPW)])


@jax.jit
def kernel(attn_weights, hidden_states, metric):
    attn_cls = attn_weights[:, :, 0, :]
    hid_t = jnp.transpose(hidden_states, (1, 0, 2))
    met_t = jnp.transpose(metric, (0, 2, 1))
    sel, idx, gidx = pl.pallas_call(
        _tc1_select,
        grid=(B,),
        in_specs=[pl.BlockSpec((B, H, S), lambda b: (0, 0, 0))],
        out_specs=[
            pl.BlockSpec((S, B), lambda b: (0, 0)),
            pl.BlockSpec((B, 8, 128), lambda b: (0, 0, 0)),
            pl.BlockSpec((B, GSLOT), lambda b: (0, 0)),
        ],
        out_shape=[
            jax.ShapeDtypeStruct((S, B), jnp.float32),
            jax.ShapeDtypeStruct((B, 8, 128), jnp.int32),
            jax.ShapeDtypeStruct((B, GSLOT), jnp.int32),
        ],
    )(attn_cls)
    table = hid_t.reshape(S * B, D)
    dom_rows = _sc_gather(gidx.reshape(NROWS), table)
    ctx_t = pl.pallas_call(
        _tc2_merge,
        grid=(B,),
        in_specs=[
            pl.BlockSpec((S, B), lambda b: (0, 0)),
            pl.BlockSpec((S, B, D), lambda b: (0, 0, 0)),
            pl.BlockSpec((B, DM, S), lambda b: (0, 0, 0)),
        ],
        out_specs=pl.BlockSpec((16, B, D), lambda b: (0, 0, 0)),
        out_shape=jax.ShapeDtypeStruct((16, B, D), jnp.float32),
    )(sel, hid_t, met_t)
    dom = dom_rows.reshape(B, GSLOT, D)[:, :NSEL, :]
    ctx = jnp.transpose(ctx_t, (1, 0, 2))[:, :CTX, :]
    out = jnp.concatenate([dom, ctx], axis=1)
    return out, idx[:, 0, :NSEL]


# speedup vs baseline: 1.5694x; 1.5694x over previous
import jax
import jax.numpy as jnp
from jax import lax
from jax.experimental import pallas as pl

B, H, S, D, DM = 8, 16, 577, 1024, 64
DOM = 54
CTX = 10
KEEP = S - (DOM + 1)
STEP = max(1, KEEP // CTX)
NSEL = DOM + 1
OUT_T = NSEL + CTX
GCOLS = 56

_HIGH = lax.Precision.HIGHEST
_DEF = lax.Precision.DEFAULT


def _mm_tn(a, bm, prec=_DEF):
    return lax.dot_general(a, bm, (((0,), (0,)), ((), ())), precision=prec)


def _mm(a, bm, prec=_DEF):
    return lax.dot_general(a, bm, (((1,), (0,)), ((), ())), precision=prec)


def _kernel(attn_cls_ref, hid_ref, met_ref, out_ref, idx_ref):
    f32 = jnp.float32
    b = pl.program_id(0)

    ii = lax.broadcasted_iota(jnp.int32, (S, S), 0)
    jj = lax.broadcasted_iota(jnp.int32, (S, S), 1)

    attn = attn_cls_ref[b]
    v = jnp.sum(attn, axis=0, keepdims=True)
    col0 = lax.broadcasted_iota(jnp.int32, (1, S), 1) == 0
    v = jnp.where(col0, -jnp.inf, v)
    vcol = _mm_tn(v, jnp.ones((1, 1), f32), _HIGH)

    vi = jnp.broadcast_to(vcol, (S, S))
    vj = jnp.broadcast_to(v, (S, S))
    beats = (vi > vj) | ((vi == vj) & (ii < jj))
    nbeat = jnp.sum(beats.astype(jnp.int32), axis=1, keepdims=True)
    rank = (S - 1) - nbeat
    selc = rank < DOM

    pp = lax.broadcasted_iota(jnp.int32, (S, 128), 1)
    in_slot = ((jnp.broadcast_to(rank, (S, 128)) == pp - 1)
               & jnp.broadcast_to(selc, (S, 128)))
    ii128 = lax.broadcasted_iota(jnp.int32, (S, 128), 0)
    idx_row = jnp.sum(jnp.where(in_slot, ii128, 0), axis=0,
                      keepdims=True)
    idx_ref[b, 0:1, :] = idx_row

    icol = lax.broadcasted_iota(jnp.int32, (S, 1), 0)
    sel_col = (selc | (icol == 0)).astype(f32)
    ltT = (jj < ii).astype(f32)
    dom_rank = _mm(ltT, sel_col)
    unsel = 1.0 - sel_col
    f_rank = icol.astype(f32) - dom_rank

    t_id = jnp.floor((f_rank + 0.5) * (1.0 / STEP))
    is_target = ((unsel > 0.5) & (f_rank == t_id * float(STEP))
                 & (f_rank < float(STEP * CTX)))
    is_merge = (unsel > 0.5) & (~is_target)

    mt = met_ref[b]
    ss = jnp.sum(mt * mt, axis=0, keepdims=True)
    mn_t = mt / jnp.sqrt(ss)
    trow = lax.broadcasted_iota(jnp.int32, (S, 16), 1).astype(f32)
    tsel = ((jnp.broadcast_to(t_id, (S, 16)) == trow)
            & jnp.broadcast_to(is_target, (S, 16))).astype(f32)
    tmat_t = _mm(mn_t, tsel, _HIGH)
    sim = _mm_tn(mn_t, tmat_t)
    sim = jnp.where(trow < float(CTX), sim, -jnp.inf)
    mx = jnp.max(sim, axis=1, keepdims=True)
    assign = jnp.min(jnp.where(sim == mx, trow, 1e9), axis=1,
                     keepdims=True)
    amat = ((jnp.broadcast_to(assign, (S, 16)) == trow)
            & jnp.broadcast_to(is_merge, (S, 16))).astype(f32)
    counts = jnp.sum(amat, axis=0, keepdims=True)
    inv_counts = 1.0 / jnp.maximum(counts, 1.0)
    ct = tsel + amat * jnp.broadcast_to(inv_counts, (S, 16))

    gr = lax.broadcasted_iota(jnp.int32, (S, GCOLS), 1).astype(f32)
    gt = ((jnp.broadcast_to(dom_rank, (S, GCOLS)) == gr)
          & jnp.broadcast_to(sel_col > 0.5, (S, GCOLS))).astype(f32)

    hid = hid_ref[:, b, :]
    out_dom = _mm_tn(gt, hid, _HIGH)
    out_ctx = _mm_tn(ct, hid)
    out_ref[0:NSEL, b, :] = out_dom[0:NSEL, :]
    out_ref[NSEL:OUT_T, b, :] = out_ctx[0:CTX, :]


@jax.jit
def kernel(attn_weights, hidden_states, metric):
    attn_cls = attn_weights[:, :, 0, :]
    hid_t = jnp.transpose(hidden_states, (1, 0, 2))
    met_t = jnp.transpose(metric, (0, 2, 1))
    out_t, idx = pl.pallas_call(
        _kernel,
        grid=(B,),
        in_specs=[
            pl.BlockSpec((B, H, S), lambda b: (0, 0, 0)),
            pl.BlockSpec((S, B, D), lambda b: (0, 0, 0)),
            pl.BlockSpec((B, DM, S), lambda b: (0, 0, 0)),
        ],
        out_specs=[
            pl.BlockSpec((OUT_T, B, D), lambda b: (0, 0, 0)),
            pl.BlockSpec((B, 8, 128), lambda b: (0, 0, 0)),
        ],
        out_shape=[
            jax.ShapeDtypeStruct((OUT_T, B, D), jnp.float32),
            jax.ShapeDtypeStruct((B, 8, 128), jnp.int32),
        ],
    )(attn_cls, hid_t, met_t)
    return jnp.transpose(out_t, (1, 0, 2)), idx[:, 0, :NSEL]
